# baseline (speedup 1.0000x reference)
import jax
import jax.numpy as jnp
from jax.experimental import pallas as pl
from jax.experimental.pallas import tpu as pltpu

_B, _C, _H, _W = 4, 384, 224, 224
_S = 16
_T = 56
_CG = 12


def _copy_kernel(x_ref, o_ref):
    o_ref[...] = x_ref[...]


def _perm_kernel(x_ref, y_ref, o_ref):
    del y_ref
    k = pl.program_id(1)

    @pl.when(k == 0)
    def _():
        o_ref[...] = x_ref[...]

    for sv in range(1, _S):
        @pl.when(k == sv)
        def _(sv=sv):
            a, r = sv // 4, sv % 4
            for j1 in range(4):
                q1 = (r + j1) % 4
                kk = (a + (r + j1) // 4) % 4
                lo, ql = j1 * _T, q1 * _T
                if kk == 0:
                    o_ref[0, :, :, lo:lo + _T] = x_ref[0, :, :, ql:ql + _T]
                else:
                    o_ref[0, :, : _H - _T * kk, lo:lo + _T] = (
                        x_ref[0, :, _T * kk:, ql:ql + _T])
                    o_ref[0, :, _H - _T * kk:, lo:lo + _T] = (
                        x_ref[0, :, : _T * kk, ql:ql + _T])


def kernel(x):
    blk = (1, _CG, _H, _W)
    y = pl.pallas_call(
        _copy_kernel,
        grid=(_B, _S),
        in_specs=[pl.BlockSpec(blk, lambda b, k: (b, k, 0, 0))],
        out_specs=pl.BlockSpec(blk, lambda b, k: (b, k, 0, 0)),
        out_shape=jax.ShapeDtypeStruct((_B, _C, _H, _W), x.dtype),
        compiler_params=pltpu.CompilerParams(
            dimension_semantics=("arbitrary", "arbitrary"),
        ),
    )(x)
    return pl.pallas_call(
        _perm_kernel,
        grid=(_B, _S),
        in_specs=[
            pl.BlockSpec(blk, lambda b, k: (b, _S + k, 0, 0)),
            pl.BlockSpec(memory_space=pltpu.MemorySpace.HBM),
        ],
        out_specs=pl.BlockSpec(blk, lambda b, k: (b, _S + k, 0, 0)),
        out_shape=jax.ShapeDtypeStruct((_B, _C, _H, _W), x.dtype),
        input_output_aliases={1: 0},
        compiler_params=pltpu.CompilerParams(
            dimension_semantics=("arbitrary", "arbitrary"),
        ),
    )(x, y)


# speedup vs baseline: 1.0501x; 1.0501x over previous
import jax
import jax.numpy as jnp
from jax.experimental import pallas as pl
from jax.experimental.pallas import tpu as pltpu

_B, _C, _H, _W = 4, 384, 224, 224
_S = 16
_T = 56
_CG = 12


def _inner_body(x0_ref, x1_ref, o0_ref, o1_ref):
    k = pl.program_id(1)
    o0_ref[...] = x0_ref[...]

    @pl.when(k == 0)
    def _():
        o1_ref[...] = x1_ref[...]

    for sv in range(1, _S):
        @pl.when(k == sv)
        def _(sv=sv):
            a, r = sv // 4, sv % 4
            for j1 in range(4):
                q1 = (r + j1) % 4
                kk = (a + (r + j1) // 4) % 4
                lo, ql = j1 * _T, q1 * _T
                if kk == 0:
                    o1_ref[0, :, :, lo:lo + _T] = x1_ref[0, :, :, ql:ql + _T]
                else:
                    o1_ref[0, :, : _H - _T * kk, lo:lo + _T] = (
                        x1_ref[0, :, _T * kk:, ql:ql + _T])
                    o1_ref[0, :, _H - _T * kk:, lo:lo + _T] = (
                        x1_ref[0, :, : _T * kk, ql:ql + _T])


def _outer(x_hbm, o_hbm):
    blk = (1, _CG, _H, _W)
    pipe = pltpu.emit_pipeline(
        _inner_body,
        grid=(_B, _S),
        in_specs=[
            pl.BlockSpec(blk, lambda b, k: (b, k, 0, 0)),
            pl.BlockSpec(blk, lambda b, k: (b, _S + k, 0, 0)),
        ],
        out_specs=[
            pl.BlockSpec(blk, lambda b, k: (b, k, 0, 0)),
            pl.BlockSpec(blk, lambda b, k: (b, _S + k, 0, 0)),
        ],
    )
    pipe(x_hbm, x_hbm, o_hbm, o_hbm)


def kernel(x):
    return pl.pallas_call(
        _outer,
        in_specs=[pl.BlockSpec(memory_space=pltpu.MemorySpace.HBM)],
        out_specs=pl.BlockSpec(memory_space=pltpu.MemorySpace.HBM),
        out_shape=jax.ShapeDtypeStruct((_B, _C, _H, _W), x.dtype),
    )(x)
